# Initial kernel scaffold; baseline (speedup 1.0000x reference)
import jax
import jax.numpy as jnp
from jax.experimental import pallas as pl


def kernel(x, token_table, pos_table):
    raise NotImplementedError("write your pallas kernel here")


# speedup vs baseline: 1.2056x; 1.2056x over previous
import jax
import jax.numpy as jnp
from jax import lax
from jax.experimental import pallas as pl
from jax.experimental.pallas import tpu as pltpu
from jax.experimental.pallas import tpu_sc as plsc

VOCAB = 1000000
MAXLEN = 200
EMBED = 32
BATCH = 4096

NC = 2
NS = 16
NW = NC * NS
N = BATCH * MAXLEN
PER_W = N // NW
CHUNK = 2 * MAXLEN
NCHUNK = PER_W // CHUNK
GATHER_SPLITS = ((0, 128), (128, 128), (256, 128), (384, 16))
L16 = EMBED // 16


def _body(x_hbm, tok_hbm, pos_hbm, out_hbm, idx_v, buf_v, pos_v, sem):
    wid = lax.axis_index("s") * NC + lax.axis_index("c")
    base0 = wid * PER_W
    pltpu.sync_copy(pos_hbm, pos_v)

    @pl.loop(0, NCHUNK)
    def _chunk(c):
        base = base0 + c * CHUNK
        pltpu.sync_copy(x_hbm.at[pl.ds(base, CHUNK)], idx_v)
        for off, sz in GATHER_SPLITS:
            pltpu.async_copy(
                tok_hbm.at[idx_v.at[pl.ds(off, sz)]],
                buf_v.at[pl.ds(off, sz)],
                sem,
            ).wait()

        @pl.loop(0, MAXLEN)
        def _add(p):
            for r in range(CHUNK // MAXLEN):
                for h in range(L16):
                    slc = (pl.ds(r * MAXLEN + p, 1), pl.ds(h * 16, 16))
                    pslc = (pl.ds(p, 1), pl.ds(h * 16, 16))
                    buf_v.at[slc][...] = buf_v.at[slc][...] + pos_v.at[pslc][...]

        pltpu.sync_copy(buf_v, out_hbm.at[pl.ds(base, CHUNK)])


def kernel(x, token_table, pos_table):
    xf = x.reshape(N).astype(jnp.int32)
    mesh = plsc.VectorSubcoreMesh(core_axis_name="c", subcore_axis_name="s")
    k = pl.kernel(
        _body,
        out_type=jax.ShapeDtypeStruct((N, EMBED), jnp.float32),
        mesh=mesh,
        compiler_params=pltpu.CompilerParams(use_tc_tiling_on_sc=False),
        scratch_types=[
            pltpu.VMEM((CHUNK,), jnp.int32),
            pltpu.VMEM((CHUNK, EMBED), jnp.float32),
            pltpu.VMEM((MAXLEN, EMBED), jnp.float32),
            pltpu.SemaphoreType.DMA,
        ],
    )
    out = k(xf, token_table, pos_table)
    return out.reshape(BATCH, MAXLEN, EMBED)


# speedup vs baseline: 1.4384x; 1.1931x over previous
import jax
import jax.numpy as jnp
from jax import lax
from jax.experimental import pallas as pl
from jax.experimental.pallas import tpu as pltpu
from jax.experimental.pallas import tpu_sc as plsc

VOCAB = 1000000
MAXLEN = 200
EMBED = 32
BATCH = 4096

NC = 2
NS = 16
NW = NC * NS
N = BATCH * MAXLEN
PER_W = N // NW
CHUNK = 4 * MAXLEN
NCHUNK = PER_W // CHUNK
ROWS_PER_P = CHUNK // MAXLEN
GATHER_SPLITS = tuple(
    (o, min(128, CHUNK - o)) for o in range(0, CHUNK, 128)
)
L16 = EMBED // 16


def _body(x_hbm, tok_hbm, pos_hbm, out_hbm,
          idx0, idx1, buf0, buf1, pos_v,
          sg0, sg1, ss0, ss1, si0, si1):
    idx = (idx0, idx1)
    buf = (buf0, buf1)
    sg = (sg0, sg1)
    ss = (ss0, ss1)
    si = (si0, si1)
    wid = lax.axis_index("s") * NC + lax.axis_index("c")
    base0 = wid * PER_W
    pltpu.sync_copy(pos_hbm, pos_v)

    def fire_gathers(b):
        for off, sz in GATHER_SPLITS:
            pltpu.async_copy(
                tok_hbm.at[idx[b].at[pl.ds(off, sz)]],
                buf[b].at[pl.ds(off, sz)],
                sg[b],
            )

    pltpu.sync_copy(x_hbm.at[pl.ds(base0, CHUNK)], idx0)
    fire_gathers(0)
    pltpu.async_copy(x_hbm.at[pl.ds(base0 + CHUNK, CHUNK)], idx1, si1)

    @pl.loop(0, NCHUNK, step=2)
    def _chunks(g):
        for b in range(2):
            i = g + b
            o = 1 - b

            @pl.when(i + 1 < NCHUNK)
            def _():
                pltpu.make_async_copy(x_hbm.at[pl.ds(0, CHUNK)], idx[o], si[o]).wait()

                @pl.when(i > 0)
                def _():
                    pltpu.make_async_copy(buf[o], out_hbm.at[pl.ds(0, CHUNK)], ss[o]).wait()

                fire_gathers(o)

            pltpu.make_async_copy(out_hbm.at[pl.ds(0, CHUNK)], buf[b], sg[b]).wait()

            @pl.when(i + 2 < NCHUNK)
            def _():
                pltpu.async_copy(
                    x_hbm.at[pl.ds(base0 + (i + 2) * CHUNK, CHUNK)], idx[b], si[b])

            @pl.loop(0, MAXLEN, unroll=2)
            def _add(p):
                for h in range(L16):
                    pv = pos_v.at[pl.ds(p, 1), pl.ds(h * 16, 16)][...]
                    for r in range(ROWS_PER_P):
                        slc = (pl.ds(r * MAXLEN + p, 1), pl.ds(h * 16, 16))
                        buf[b].at[slc][...] = buf[b].at[slc][...] + pv

            pltpu.async_copy(buf[b], out_hbm.at[pl.ds(base0 + i * CHUNK, CHUNK)], ss[b])

    pltpu.make_async_copy(buf0, out_hbm.at[pl.ds(0, CHUNK)], ss0).wait()
    pltpu.make_async_copy(buf1, out_hbm.at[pl.ds(0, CHUNK)], ss1).wait()


def kernel(x, token_table, pos_table):
    xf = x.reshape(N).astype(jnp.int32)
    mesh = plsc.VectorSubcoreMesh(core_axis_name="c", subcore_axis_name="s")
    k = pl.kernel(
        _body,
        out_type=jax.ShapeDtypeStruct((N, EMBED), jnp.float32),
        mesh=mesh,
        compiler_params=pltpu.CompilerParams(use_tc_tiling_on_sc=False),
        scratch_types=[
            pltpu.VMEM((CHUNK,), jnp.int32),
            pltpu.VMEM((CHUNK,), jnp.int32),
            pltpu.VMEM((CHUNK, EMBED), jnp.float32),
            pltpu.VMEM((CHUNK, EMBED), jnp.float32),
            pltpu.VMEM((MAXLEN, EMBED), jnp.float32),
            pltpu.SemaphoreType.DMA,
            pltpu.SemaphoreType.DMA,
            pltpu.SemaphoreType.DMA,
            pltpu.SemaphoreType.DMA,
            pltpu.SemaphoreType.DMA,
            pltpu.SemaphoreType.DMA,
        ],
    )
    out = k(xf, token_table, pos_table)
    return out.reshape(BATCH, MAXLEN, EMBED)
